# baseline (speedup 1.0000x reference)
import functools

import jax
import jax.numpy as jnp
from jax import lax
from jax.experimental import pallas as pl
from jax.experimental.pallas import tpu as pltpu
from jax.experimental.pallas import tpu_sc as plsc

EMB = 768
LANES = 16
NVREG = EMB // LANES
NC, NS = 2, 16
NW = NC * NS
EPS = 1e-12
CH = 128


def _sc_embed(tokens_flat, tt_flat, word, ttemb, pos, gamma, beta, seq_len):
    n = tokens_flat.shape[0]
    per_w = n // NW
    n_chunks = per_w // CH

    mesh = plsc.VectorSubcoreMesh(core_axis_name="c", subcore_axis_name="s")

    @functools.partial(
        pl.kernel,
        mesh=mesh,
        out_type=jax.ShapeDtypeStruct((n, EMB), jnp.float32),
        scratch_types=[
            pltpu.VMEM((CH,), jnp.int32),
            pltpu.VMEM((CH,), jnp.int32),
            pltpu.VMEM((CH, EMB), jnp.float32),
            pltpu.VMEM((EMB,), jnp.float32),
            pltpu.VMEM((EMB,), jnp.float32),
            pltpu.SemaphoreType.DMA,
        ],
        compiler_params=pltpu.CompilerParams(use_tc_tiling_on_sc=False, needs_layout_passes=False),
    )
    def k(tok_hbm, tt_hbm, word_hbm, ttemb_hbm, pos_hbm, gamma_hbm, beta_hbm,
          out_hbm, idx_v, tti_v, buf, gamma_v, beta_v, sem):
        wid = lax.axis_index("s") * NC + lax.axis_index("c")
        pltpu.sync_copy(gamma_hbm, gamma_v)
        pltpu.sync_copy(beta_hbm, beta_v)
        inv_n = 1.0 / EMB

        def ln_group(g, carry):
            rows = g * LANES + lax.iota(jnp.int32, LANES)

            def p1(c, ca):
                acc, acc2 = ca
                cols = jnp.full((LANES,), c, jnp.int32)
                v = plsc.load_gather(buf, [rows, cols])
                return (acc + v, acc2 + v * v)

            acc, acc2 = lax.fori_loop(
                0, EMB, p1,
                (jnp.zeros((LANES,), jnp.float32),
                 jnp.zeros((LANES,), jnp.float32)))
            mean = acc * inv_n
            x = acc2 * inv_n - mean * mean + EPS
            i = lax.bitcast_convert_type(x, jnp.int32)
            i = jnp.int32(0x5F3759DF) - lax.shift_right_arithmetic(i, 1)
            y = lax.bitcast_convert_type(i, jnp.float32)
            for _ in range(3):
                y = y * (1.5 - 0.5 * x * y * y)

            def p2(c, cc):
                cols = jnp.full((LANES,), c, jnp.int32)
                v = plsc.load_gather(buf, [rows, cols])
                gv = plsc.load_gather(gamma_v, [cols])
                bv = plsc.load_gather(beta_v, [cols])
                v = (v - mean) * y * gv + bv
                plsc.store_scatter(buf, [rows, cols], v)
                return cc

            return lax.fori_loop(0, EMB, p2, carry)

        for c in range(n_chunks):
            base = wid * per_w + c * CH
            pbase = lax.rem(base, seq_len)
            pltpu.sync_copy(tok_hbm.at[pl.ds(base, CH)], idx_v)
            pltpu.sync_copy(tt_hbm.at[pl.ds(base, CH)], tti_v)
            pltpu.sync_copy(pos_hbm.at[pl.ds(pbase, CH)], buf)
            pltpu.async_copy(ttemb_hbm.at[tti_v], buf, sem, add=True).wait()
            pltpu.async_copy(word_hbm.at[idx_v], buf, sem, add=True).wait()
            lax.fori_loop(0, CH // LANES, ln_group, 0)
            pltpu.sync_copy(buf, out_hbm.at[pl.ds(base, CH)])

    return k(tokens_flat, tt_flat, word, ttemb, pos, gamma, beta)


def kernel(tokens, tokens_type, word_embedding, token_type_embedding,
           position_embedding, ln_gamma, ln_beta):
    B, L = tokens.shape
    tokens_flat = tokens.reshape(-1).astype(jnp.int32)
    tt_flat = tokens_type.reshape(-1).astype(jnp.int32)
    out = _sc_embed(tokens_flat, tt_flat, word_embedding,
                    token_type_embedding, position_embedding,
                    ln_gamma, ln_beta, L)
    return out.reshape(B, L, EMB)


# speedup vs baseline: 1.5281x; 1.5281x over previous
import functools

import jax
import jax.numpy as jnp
from jax import lax
from jax.experimental import pallas as pl
from jax.experimental.pallas import tpu as pltpu
from jax.experimental.pallas import tpu_sc as plsc

EMB = 768
LANES = 16
NVREG = EMB // LANES
NC, NS = 2, 16
NW = NC * NS
EPS = 1e-12
CH = 128


def _sc_embed(tokens_flat, tt_flat, word, ttemb, pos, gamma, beta, seq_len):
    n = tokens_flat.shape[0]
    per_w = n // NW
    n_chunks = per_w // CH

    mesh = plsc.VectorSubcoreMesh(core_axis_name="c", subcore_axis_name="s")

    @functools.partial(
        pl.kernel,
        mesh=mesh,
        out_type=jax.ShapeDtypeStruct((n, EMB), jnp.float32),
        scratch_types=[
            pltpu.VMEM((CH,), jnp.int32),
            pltpu.VMEM((CH,), jnp.int32),
            pltpu.VMEM((CH, EMB), jnp.float32),
            pltpu.VMEM((EMB,), jnp.float32),
            pltpu.VMEM((EMB,), jnp.float32),
            pltpu.SemaphoreType.DMA,
        ],
        compiler_params=pltpu.CompilerParams(use_tc_tiling_on_sc=False, needs_layout_passes=False),
    )
    def k(tok_hbm, tt_hbm, word_hbm, ttemb_hbm, pos_hbm, gamma_hbm, beta_hbm,
          out_hbm, idx_v, tti_v, buf, gamma_v, beta_v, sem):
        wid = lax.axis_index("s") * NC + lax.axis_index("c")
        pltpu.sync_copy(gamma_hbm, gamma_v)
        pltpu.sync_copy(beta_hbm, beta_v)
        inv_n = 1.0 / EMB

        def ln_body(t, carry):
            vs = []
            acc = jnp.zeros((LANES,), jnp.float32)
            acc2 = jnp.zeros((LANES,), jnp.float32)
            for j in range(NVREG):
                v = buf[t, pl.ds(j * LANES, LANES)]
                vs.append(v)
                acc = acc + v
                acc2 = acc2 + v * v
            mean = jnp.sum(acc) * inv_n
            var = jnp.sum(acc2) * inv_n - mean * mean
            x = jnp.full((LANES,), var + EPS, jnp.float32)
            mean_v = jnp.full((LANES,), mean, jnp.float32)
            i = lax.bitcast_convert_type(x, jnp.int32)
            i = jnp.int32(0x5F3759DF) - lax.shift_right_arithmetic(i, 1)
            y = lax.bitcast_convert_type(i, jnp.float32)
            for _ in range(3):
                y = y * (1.5 - 0.5 * x * y * y)
            for j in range(NVREG):
                sl = pl.ds(j * LANES, LANES)
                buf[t, sl] = ((vs[j] - mean_v) * y * gamma_v[sl]
                              + beta_v[sl])
            return carry

        for c in range(n_chunks):
            base = wid * per_w + c * CH
            pbase = lax.rem(base, seq_len)
            pltpu.sync_copy(tok_hbm.at[pl.ds(base, CH)], idx_v)
            pltpu.sync_copy(tt_hbm.at[pl.ds(base, CH)], tti_v)
            pltpu.sync_copy(pos_hbm.at[pl.ds(pbase, CH)], buf)
            pltpu.async_copy(ttemb_hbm.at[tti_v], buf, sem, add=True).wait()
            pltpu.async_copy(word_hbm.at[idx_v], buf, sem, add=True).wait()
            lax.fori_loop(0, CH, ln_body, 0)
            pltpu.sync_copy(buf, out_hbm.at[pl.ds(base, CH)])

    return k(tokens_flat, tt_flat, word, ttemb, pos, gamma, beta)


def kernel(tokens, tokens_type, word_embedding, token_type_embedding,
           position_embedding, ln_gamma, ln_beta):
    B, L = tokens.shape
    tokens_flat = tokens.reshape(-1).astype(jnp.int32)
    tt_flat = tokens_type.reshape(-1).astype(jnp.int32)
    out = _sc_embed(tokens_flat, tt_flat, word_embedding,
                    token_type_embedding, position_embedding,
                    ln_gamma, ln_beta, L)
    return out.reshape(B, L, EMB)
